# baseline (speedup 1.0000x reference)
import functools

import jax
import jax.numpy as jnp
from jax import lax
from jax.experimental import pallas as pl
from jax.experimental.pallas import tpu as pltpu
from jax.experimental.pallas import tpu_sc as plsc

N = 50000
D = 64
HD = 32
E = 800000
E_PAD = 819200
NC = 2
NS = 16


_EDGES_PER_TILE = E_PAD // NS
_CHUNK = 1024
_NCHUNK = _EDGES_PER_TILE // _CHUNK
NP = 50048
_ROWS_PER_TILE = NP // NS


def _spmm_body(row_hbm, col_hbm, w_hbm, ego_hbm, msg_hbm,
               slab, rows_g, rows_v, colbuf, gidxbuf, rowbuf, wbuf, sem):
    c = lax.axis_index("c")
    s = lax.axis_index("s")

    def _z(i, _):
        rows_v[i, pl.ds(0, 16)] = jnp.zeros((16,), jnp.float32)
        rows_v[i, pl.ds(16, 16)] = jnp.zeros((16,), jnp.float32)
        return 0
    lax.fori_loop(0, 128, _z, 0)
    zrow = pl.multiple_of(s * _ROWS_PER_TILE, 8)
    for k in range(24):
        pltpu.sync_copy(rows_v,
                        slab.at[pl.ds(pl.multiple_of(zrow + k * 128, 8), 128)])
    pltpu.sync_copy(rows_v.at[pl.ds(0, 56)],
                    slab.at[pl.ds(pl.multiple_of(zrow + 3072, 8), 56)])
    plsc.subcore_barrier()

    ebase = s * _EDGES_PER_TILE

    def _chunk(ci, _):
        r0 = pl.multiple_of((ebase + ci * _CHUNK) // 128, 8)
        pltpu.sync_copy(col_hbm.at[pl.ds(r0, 8)], colbuf)
        pltpu.sync_copy(row_hbm.at[pl.ds(r0, 8)], rowbuf)
        pltpu.sync_copy(w_hbm.at[pl.ds(r0, 8)], wbuf)
        for a in range(8):
            def _gi(k, _):
                v = colbuf[a, pl.ds(k * 16, 16)]
                gidxbuf[a, pl.ds(k * 16, 16)] = v + v + c
                return 0
            lax.fori_loop(0, 8, _gi, 0)
        for h in range(2):
            cps = []
            for a in range(4):
                cps.append(pltpu.async_copy(ego_hbm.at[gidxbuf.at[h * 4 + a]],
                                            rows_g.at[pl.ds(a * 128, 128)],
                                            sem))
            for cp in cps:
                cp.wait()
            for a in range(4):
                def _sc(g, _):
                    w16 = wbuf[h * 4 + a, pl.ds(g * 16, 16)]
                    for l in range(16):
                        w = w16[l]
                        r = g * 16 + l
                        u = plsc.bitcast(rows_g[a * 128 + r, pl.ds(0, 16)],
                                         jnp.bfloat16)
                        ua, ub = plsc.unpack(
                            u, format=plsc.PackFormat.INTERLEAVED)
                        rows_v[r, pl.ds(0, 16)] = ua * w
                        rows_v[r, pl.ds(16, 16)] = ub * w
                    return 0
                lax.fori_loop(0, 8, _sc, 0)
                pltpu.sync_copy(rows_v,
                                slab.at[rowbuf.at[h * 4 + a]], add=True)
        return 0

    lax.fori_loop(0, _NCHUNK, _chunk, 0)
    plsc.subcore_barrier()
    pltpu.sync_copy(slab.at[pl.ds(zrow, _ROWS_PER_TILE)],
                    msg_hbm.at[c, pl.ds(zrow, _ROWS_PER_TILE)])


def _spmm(row2d, col2d, w2d, ego_r):
    mesh = plsc.VectorSubcoreMesh(core_axis_name="c", subcore_axis_name="s")
    k = pl.kernel(
        _spmm_body,
        mesh=mesh,
        compiler_params=pltpu.CompilerParams(use_tc_tiling_on_sc=False,
                                             needs_layout_passes=False),
        out_type=jax.ShapeDtypeStruct((NC, NP, HD), jnp.float32),
        scratch_types=[
            pltpu.VMEM_SHARED((NP, HD), jnp.float32),
            pltpu.VMEM((_CHUNK // 2, 16), jnp.int32),
            pltpu.VMEM((128, HD), jnp.float32),
            pltpu.VMEM((8, 128), jnp.int32),
            pltpu.VMEM((8, 128), jnp.int32),
            pltpu.VMEM((8, 128), jnp.int32),
            pltpu.VMEM((8, 128), jnp.float32),
            pltpu.SemaphoreType.DMA,
        ],
    )
    return k(row2d, col2d, w2d, ego_r)


_BN = 1000


def _dense_body(m_ref, ego_ref, w1t_ref, b1_ref, w2t_ref, b2_ref,
                ego_out_ref, nrm_out_ref):
    msg = jnp.concatenate([m_ref[0], m_ref[1]], axis=-1)
    ego = ego_ref[...]
    aggr = jnp.dot(msg, w1t_ref[...], preferred_element_type=jnp.float32)
    aggr = aggr + b1_ref[...]
    bi = jnp.dot(ego * msg, w2t_ref[...], preferred_element_type=jnp.float32)
    bi = bi + b2_ref[...]
    x = aggr + bi
    y = jnp.where(x >= 0, x, 0.2 * x)
    ego_out_ref[...] = y
    nrm = jnp.sqrt(jnp.sum(y * y, axis=1, keepdims=True))
    nrm_out_ref[...] = (y / jnp.maximum(nrm, 1e-12)).astype(jnp.bfloat16)


def _dense(msg2, ego, w1t, b1r, w2t, b2r):
    return pl.pallas_call(
        _dense_body,
        grid=(N // _BN,),
        in_specs=[
            pl.BlockSpec((NC, _BN, HD), lambda i: (0, i, 0)),
            pl.BlockSpec((_BN, D), lambda i: (i, 0)),
            pl.BlockSpec((D, D), lambda i: (0, 0)),
            pl.BlockSpec((1, D), lambda i: (0, 0)),
            pl.BlockSpec((D, D), lambda i: (0, 0)),
            pl.BlockSpec((1, D), lambda i: (0, 0)),
        ],
        out_specs=[
            pl.BlockSpec((_BN, D), lambda i: (i, 0)),
            pl.BlockSpec((_BN, D), lambda i: (i, 0)),
        ],
        out_shape=[
            jax.ShapeDtypeStruct((N, D), jnp.float32),
            jax.ShapeDtypeStruct((N, D), jnp.bfloat16),
        ],
    )(msg2, ego, w1t, b1r, w2t, b2r)


_RD = 4 * D
_EDGES_PER_WORKER = E_PAD // (NC * NS)
_K3_CHUNK = 64
_K3_NCHUNK = _EDGES_PER_WORKER // _K3_CHUNK
_GROW = 2 * _K3_CHUNK


def _dots_body(idx2_hbm, res_hbm, out_hbm,
               ibuf, buf0, buf1, acc, outbuf, sem0, sem1):
    c = lax.axis_index("c")
    s = lax.axis_index("s")
    wid = s * NC + c
    wbase = wid * _EDGES_PER_WORKER

    pltpu.sync_copy(
        idx2_hbm.at[pl.ds(pl.multiple_of(wid * _K3_NCHUNK * _GROW, 8),
                          _K3_NCHUNK * _GROW)], ibuf)

    bufs = [buf0, buf1]
    sems = [sem0, sem1]

    def _issue(g, b):
        off = pl.multiple_of(g * _GROW, 8)
        pltpu.async_copy(res_hbm.at[ibuf.at[pl.ds(off, _GROW)]],
                         bufs[b], sems[b])

    def _drain(b):
        pltpu.make_async_copy(res_hbm.at[pl.ds(0, _GROW)],
                              bufs[b], sems[b]).wait()

    def _compute(g, b):
        buf = bufs[b]

        def _edge(e, _):
            a = None
            for k in range(_RD // 32):
                u = plsc.bitcast(buf[e, pl.ds(k * 16, 16)], jnp.bfloat16)
                v = plsc.bitcast(buf[_K3_CHUNK + e, pl.ds(k * 16, 16)],
                                 jnp.bfloat16)
                ua, ub = plsc.unpack(u, format=plsc.PackFormat.INTERLEAVED)
                va, vb = plsc.unpack(v, format=plsc.PackFormat.INTERLEAVED)
                t = ua * va + ub * vb
                a = t if a is None else a + t
            acc[e, pl.ds(0, 16)] = a
            return 0
        lax.fori_loop(0, _K3_CHUNK, _edge, 0)

        def _red(gg, _):
            rows = gg * 16 + lax.iota(jnp.int32, 16)
            t = plsc.load_gather(acc, [rows, jnp.full((16,), 0, jnp.int32)])
            for l in range(1, 16):
                t = t + plsc.load_gather(acc, [rows, jnp.full((16,), l, jnp.int32)])
            outbuf[pl.ds(gg * 16, 16)] = t
            return 0
        lax.fori_loop(0, _K3_CHUNK // 16, _red, 0)

        base = pl.multiple_of(wbase + g * _K3_CHUNK, 8)
        pltpu.sync_copy(outbuf, out_hbm.at[pl.ds(base, _K3_CHUNK)])

    _issue(0, 0)
    _issue(1, 1)

    def _pair(p, _):
        for b in range(2):
            g = p * 2 + b
            _drain(b)
            _compute(g, b)

            @pl.when(g + 2 < _K3_NCHUNK)
            def _():
                _issue(g + 2, b)
        return 0

    lax.fori_loop(0, _K3_NCHUNK // 2, _pair, 0)


def _edge_dots(idx2, res):
    mesh = plsc.VectorSubcoreMesh(core_axis_name="c", subcore_axis_name="s")
    k = pl.kernel(
        _dots_body,
        mesh=mesh,
        compiler_params=pltpu.CompilerParams(needs_layout_passes=False),
        out_type=jax.ShapeDtypeStruct((E_PAD,), jnp.float32),
        scratch_types=[
            pltpu.VMEM((_K3_NCHUNK * _GROW,), jnp.int32),
            pltpu.VMEM((_GROW, _RD // 2), jnp.int32),
            pltpu.VMEM((_GROW, _RD // 2), jnp.int32),
            pltpu.VMEM((_K3_CHUNK, 16), jnp.float32),
            pltpu.VMEM((_K3_CHUNK,), jnp.float32),
            pltpu.SemaphoreType.DMA,
            pltpu.SemaphoreType.DMA,
        ],
    )
    return k(idx2, res)


def _shuf(x):
    b = x.astype(jnp.bfloat16).reshape(N, 2, 2, 16)
    b = jnp.transpose(b, (0, 1, 3, 2))
    w = jax.lax.bitcast_convert_type(b.reshape(N, 32, 2), jnp.int32)
    return w.reshape(2 * N, 16)


def kernel(edge_index, edge_weight, emb, W1, b1, W2, b2):
    row = edge_index[0].astype(jnp.int32)
    col = edge_index[1].astype(jnp.int32)
    pad = E_PAD - E
    row_p = jnp.pad(row, (0, pad))
    col_p = jnp.pad(col, (0, pad))
    w_p = jnp.pad(edge_weight, (0, pad))
    row2d = row_p.reshape(E_PAD // 128, 128)
    col2d = col_p.reshape(E_PAD // 128, 128)
    w2d = w_p.reshape(E_PAD // 128, 128)

    w1t = jnp.transpose(W1, (0, 2, 1))
    w2t = jnp.transpose(W2, (0, 2, 1))
    b1r = b1.reshape(W1.shape[0], 1, D)
    b2r = b2.reshape(W1.shape[0], 1, D)

    ego = emb
    tables = [emb.astype(jnp.bfloat16)]
    for i in range(W1.shape[0]):
        ego_r = _shuf(ego)
        msg2 = _spmm(row2d, col2d, w2d, ego_r)[:, :N, :]
        ego, normed = _dense(msg2, ego, w1t[i], b1r[i], w2t[i], b2r[i])
        tables.append(normed)

    res = jnp.concatenate(tables, axis=1)
    res32 = jax.lax.bitcast_convert_type(res.reshape(N, _RD // 2, 2),
                                         jnp.int32)
    idx2 = jnp.concatenate([row_p.reshape(-1, _K3_CHUNK),
                            col_p.reshape(-1, _K3_CHUNK)], axis=1).reshape(-1)
    out = _edge_dots(idx2, res32)
    return out[:E]


# speedup vs baseline: 1.2674x; 1.2674x over previous
import functools

import jax
import jax.numpy as jnp
from jax import lax
from jax.experimental import pallas as pl
from jax.experimental.pallas import tpu as pltpu
from jax.experimental.pallas import tpu_sc as plsc

N = 50000
D = 64
HD = 32
E = 800000
E_PAD = 819200
NC = 2
NS = 16


_EDGES_PER_TILE = E_PAD // NS
_CHUNK = 1024
_NCHUNK = _EDGES_PER_TILE // _CHUNK
NP = 50048
_ROWS_PER_TILE = NP // NS


def _spmm_body(row_hbm, col_hbm, w_hbm, ego_hbm, msg_hbm,
               slab, rows_v, colbuf, gidxbuf, rowbuf, wbuf, zbuf, sem):
    c = lax.axis_index("c")
    s = lax.axis_index("s")

    def _z(i, _):
        zbuf[i, pl.ds(0, 16)] = jnp.zeros((16,), jnp.float32)
        zbuf[i, pl.ds(16, 16)] = jnp.zeros((16,), jnp.float32)
        return 0
    lax.fori_loop(0, 128, _z, 0)
    zrow = pl.multiple_of(s * _ROWS_PER_TILE, 8)
    for k in range(24):
        pltpu.sync_copy(zbuf,
                        slab.at[pl.ds(pl.multiple_of(zrow + k * 128, 8), 128)])
    pltpu.sync_copy(zbuf.at[pl.ds(0, 56)],
                    slab.at[pl.ds(pl.multiple_of(zrow + 3072, 8), 56)])
    plsc.subcore_barrier()

    ebase = s * _EDGES_PER_TILE

    def _chunk(ci, _):
        r0 = pl.multiple_of((ebase + ci * _CHUNK) // 128, 8)
        pltpu.sync_copy(col_hbm.at[pl.ds(r0, 8)], colbuf)
        pltpu.sync_copy(row_hbm.at[pl.ds(r0, 8)], rowbuf)
        pltpu.sync_copy(w_hbm.at[pl.ds(r0, 8)], wbuf)
        for a in range(8):
            def _gi(k, _):
                v = colbuf[a, pl.ds(k * 16, 16)]
                gidxbuf[a, pl.ds(k * 16, 16)] = v + v + c
                return 0
            lax.fori_loop(0, 8, _gi, 0)
        for h in range(2):
            cps = []
            for a in range(4):
                cps.append(pltpu.async_copy(ego_hbm.at[gidxbuf.at[h * 4 + a]],
                                            rows_v.at[pl.ds(a * 128, 128)],
                                            sem))
            for cp in cps:
                cp.wait()
            for a in range(4):
                def _sc(g, _):
                    w16 = wbuf[h * 4 + a, pl.ds(g * 16, 16)]
                    for l in range(16):
                        w = w16[l]
                        r = a * 128 + g * 16 + l
                        v0 = rows_v[r, pl.ds(0, 16)]
                        rows_v[r, pl.ds(0, 16)] = v0 * w
                        v1 = rows_v[r, pl.ds(16, 16)]
                        rows_v[r, pl.ds(16, 16)] = v1 * w
                    return 0
                lax.fori_loop(0, 8, _sc, 0)
            for a in range(4):
                pltpu.sync_copy(rows_v.at[pl.ds(a * 128, 128)],
                                slab.at[rowbuf.at[h * 4 + a]], add=True)
        return 0

    lax.fori_loop(0, _NCHUNK, _chunk, 0)
    plsc.subcore_barrier()
    pltpu.sync_copy(slab.at[pl.ds(zrow, _ROWS_PER_TILE)],
                    msg_hbm.at[c, pl.ds(zrow, _ROWS_PER_TILE)])


def _spmm(row2d, col2d, w2d, ego_r):
    mesh = plsc.VectorSubcoreMesh(core_axis_name="c", subcore_axis_name="s")
    k = pl.kernel(
        _spmm_body,
        mesh=mesh,
        compiler_params=pltpu.CompilerParams(use_tc_tiling_on_sc=False),
        out_type=jax.ShapeDtypeStruct((NC, NP, HD), jnp.float32),
        scratch_types=[
            pltpu.VMEM_SHARED((NP, HD), jnp.float32),
            pltpu.VMEM((_CHUNK // 2, HD), jnp.float32),
            pltpu.VMEM((8, 128), jnp.int32),
            pltpu.VMEM((8, 128), jnp.int32),
            pltpu.VMEM((8, 128), jnp.int32),
            pltpu.VMEM((8, 128), jnp.float32),
            pltpu.VMEM((128, HD), jnp.float32),
            pltpu.SemaphoreType.DMA,
        ],
    )
    return k(row2d, col2d, w2d, ego_r)


_BN = 1000


def _dense_body(m_ref, ego_ref, w1t_ref, b1_ref, w2t_ref, b2_ref,
                ego_out_ref, pk_out_ref):
    msg = jnp.concatenate([m_ref[0], m_ref[1]], axis=-1)
    ego = ego_ref[...]
    aggr = jnp.dot(msg, w1t_ref[...], preferred_element_type=jnp.float32)
    aggr = aggr + b1_ref[...]
    bi = jnp.dot(ego * msg, w2t_ref[...], preferred_element_type=jnp.float32)
    bi = bi + b2_ref[...]
    x = aggr + bi
    y = jnp.where(x >= 0, x, 0.2 * x)
    ego_out_ref[...] = y
    nrm = jnp.sqrt(jnp.sum(y * y, axis=1, keepdims=True))
    nb = y / jnp.maximum(nrm, 1e-12)
    a = jax.lax.bitcast_convert_type(nb[:, :HD], jnp.int32)
    b = jax.lax.bitcast_convert_type(nb[:, HD:], jnp.int32)
    a = a + 0x7FFF + ((a >> 16) & 1)
    b = b + 0x7FFF + ((b >> 16) & 1)
    lo = (a >> 16) & 0xFFFF
    hi = b & jnp.int32(-65536)
    pk_out_ref[...] = lo | hi


def _dense(msg2, ego, w1t, b1r, w2t, b2r):
    return pl.pallas_call(
        _dense_body,
        grid=(N // _BN,),
        in_specs=[
            pl.BlockSpec((NC, _BN, HD), lambda i: (0, i, 0)),
            pl.BlockSpec((_BN, D), lambda i: (i, 0)),
            pl.BlockSpec((D, D), lambda i: (0, 0)),
            pl.BlockSpec((1, D), lambda i: (0, 0)),
            pl.BlockSpec((D, D), lambda i: (0, 0)),
            pl.BlockSpec((1, D), lambda i: (0, 0)),
        ],
        out_specs=[
            pl.BlockSpec((_BN, D), lambda i: (i, 0)),
            pl.BlockSpec((_BN, HD), lambda i: (i, 0)),
        ],
        out_shape=[
            jax.ShapeDtypeStruct((N, D), jnp.float32),
            jax.ShapeDtypeStruct((N, HD), jnp.int32),
        ],
    )(msg2, ego, w1t, b1r, w2t, b2r)


_RD = 4 * D
_EDGES_PER_WORKER = E_PAD // (NC * NS)
_K3_CHUNK = 64
_K3_NCHUNK = _EDGES_PER_WORKER // _K3_CHUNK
_GROW = 2 * _K3_CHUNK


def _dots_body(idx2_hbm, res_hbm, out_hbm,
               ibuf, buf0, buf1, acc, outbuf, sem0, sem1):
    c = lax.axis_index("c")
    s = lax.axis_index("s")
    wid = s * NC + c
    wbase = wid * _EDGES_PER_WORKER

    pltpu.sync_copy(
        idx2_hbm.at[pl.ds(pl.multiple_of(wid * _K3_NCHUNK * _GROW, 8),
                          _K3_NCHUNK * _GROW)], ibuf)

    bufs = [buf0, buf1]
    sems = [sem0, sem1]

    def _issue(g, b):
        off = pl.multiple_of(g * _GROW, 8)
        pltpu.async_copy(res_hbm.at[ibuf.at[pl.ds(off, _GROW)]],
                         bufs[b], sems[b])

    def _drain(b):
        pltpu.make_async_copy(res_hbm.at[pl.ds(0, _GROW)],
                              bufs[b], sems[b]).wait()

    def _compute(g, b):
        buf = bufs[b]

        def _edge(e, _):
            a = None
            for k in range(_RD // 32):
                u = plsc.bitcast(buf[e, pl.ds(k * 16, 16)], jnp.bfloat16)
                v = plsc.bitcast(buf[_K3_CHUNK + e, pl.ds(k * 16, 16)],
                                 jnp.bfloat16)
                ua, ub = plsc.unpack(u, format=plsc.PackFormat.INTERLEAVED)
                va, vb = plsc.unpack(v, format=plsc.PackFormat.INTERLEAVED)
                t = ua * va + ub * vb
                a = t if a is None else a + t
            acc[e, pl.ds(0, 16)] = a
            return 0
        lax.fori_loop(0, _K3_CHUNK, _edge, 0)

        def _red(gg, _):
            rows = gg * 16 + lax.iota(jnp.int32, 16)
            t = plsc.load_gather(acc, [rows, jnp.full((16,), 0, jnp.int32)])
            for l in range(1, 16):
                t = t + plsc.load_gather(acc, [rows, jnp.full((16,), l, jnp.int32)])
            outbuf[pl.ds(gg * 16, 16)] = t
            return 0
        lax.fori_loop(0, _K3_CHUNK // 16, _red, 0)

        base = pl.multiple_of(wbase + g * _K3_CHUNK, 8)
        pltpu.sync_copy(outbuf, out_hbm.at[pl.ds(base, _K3_CHUNK)])

    _issue(0, 0)
    _issue(1, 1)

    def _pair(p, _):
        for b in range(2):
            g = p * 2 + b
            _drain(b)
            _compute(g, b)

            @pl.when(g + 2 < _K3_NCHUNK)
            def _():
                _issue(g + 2, b)
        return 0

    lax.fori_loop(0, _K3_NCHUNK // 2, _pair, 0)


def _edge_dots(idx2, res):
    mesh = plsc.VectorSubcoreMesh(core_axis_name="c", subcore_axis_name="s")
    k = pl.kernel(
        _dots_body,
        mesh=mesh,
        compiler_params=pltpu.CompilerParams(needs_layout_passes=False),
        out_type=jax.ShapeDtypeStruct((E_PAD,), jnp.float32),
        scratch_types=[
            pltpu.VMEM((_K3_NCHUNK * _GROW,), jnp.int32),
            pltpu.VMEM((_GROW, _RD // 2), jnp.int32),
            pltpu.VMEM((_GROW, _RD // 2), jnp.int32),
            pltpu.VMEM((_K3_CHUNK, 16), jnp.float32),
            pltpu.VMEM((_K3_CHUNK,), jnp.float32),
            pltpu.SemaphoreType.DMA,
            pltpu.SemaphoreType.DMA,
        ],
    )
    return k(idx2, res)


def kernel(edge_index, edge_weight, emb, W1, b1, W2, b2):
    row = edge_index[0].astype(jnp.int32)
    col = edge_index[1].astype(jnp.int32)
    pad = E_PAD - E
    row_p = jnp.pad(row, (0, pad))
    col_p = jnp.pad(col, (0, pad))
    w_p = jnp.pad(edge_weight, (0, pad))
    row2d = row_p.reshape(E_PAD // 128, 128)
    col2d = col_p.reshape(E_PAD // 128, 128)
    w2d = w_p.reshape(E_PAD // 128, 128)

    w1t = jnp.transpose(W1, (0, 2, 1))
    w2t = jnp.transpose(W2, (0, 2, 1))
    b1r = b1.reshape(W1.shape[0], 1, D)
    b2r = b2.reshape(W1.shape[0], 1, D)

    ego = emb
    emb_w = jax.lax.bitcast_convert_type(
        emb.astype(jnp.bfloat16).reshape(N, HD, 2), jnp.int32)
    tables = [emb_w]
    for i in range(W1.shape[0]):
        ego_r = ego.reshape(2 * N, HD)
        msg2 = _spmm(row2d, col2d, w2d, ego_r)[:, :N, :]
        ego, packed = _dense(msg2, ego, w1t[i], b1r[i], w2t[i], b2r[i])
        tables.append(packed)

    res32 = jnp.concatenate(tables, axis=1)
    idx2 = jnp.concatenate([row_p.reshape(-1, _K3_CHUNK),
                            col_p.reshape(-1, _K3_CHUNK)], axis=1).reshape(-1)
    out = _edge_dots(idx2, res32)
    return out[:E]
